# baseline (speedup 1.0000x reference)
import functools
import jax
import jax.numpy as jnp
from jax import lax
from jax.experimental import pallas as pl
from jax.experimental.pallas import tpu as pltpu
from jax.experimental.pallas import tpu_sc as plsc

_ROWS = 1024
_COLS = 100000
_NW = 32
_RPW = _ROWS // _NW
_NG = _RPW // 8
_CH = 5888
_NC = 17
_VPB = 46
_NB = 8
_TVPB = 40
_NT = _NG * _NC

def _bmax(x, rbf):
    for sh in (8, 4, 2, 1):
        rbf[pl.ds(0, 16)] = x
        rbf[pl.ds(16, 16)] = x
        x = jnp.maximum(x, rbf[pl.ds(sh, 16)])
    return x


def _bmin(x, rbi):
    for sh in (8, 4, 2, 1):
        rbi[pl.ds(0, 16)] = x
        rbi[pl.ds(16, 16)] = x
        x = jnp.minimum(x, rbi[pl.ds(sh, 16)])
    return x


def _pop_max(U, iota, neg, rbf, rbi):
    m = _bmax(U[0], rbf)
    ic = jnp.where(U[0] == m, iota, 16)
    f = _bmin(ic, rbi)
    hit = iota == f
    U = [jnp.where(hit, U[1], U[0]),
         jnp.where(hit, U[2], U[1]),
         jnp.where(hit, U[3], U[2]),
         jnp.where(hit, U[4], U[3]),
         jnp.where(hit, neg, U[4])]
    return m, U


def _expf(x):
    x = jnp.minimum(jnp.maximum(x, jnp.float32(-87.0)), jnp.float32(87.0))
    t = x * jnp.float32(1.4426950408889634)
    magic = jnp.float32(12582912.0)
    fm = t + magic
    k = lax.bitcast_convert_type(fm, jnp.int32) - jnp.int32(0x4B400000)
    tk = fm - magic
    r = (x - tk * jnp.float32(0.693359375)) - tk * jnp.float32(-2.12194440e-4)
    p = jnp.float32(1.0 / 720.0)
    p = p * r + jnp.float32(1.0 / 120.0)
    p = p * r + jnp.float32(1.0 / 24.0)
    p = p * r + jnp.float32(1.0 / 6.0)
    p = p * r + jnp.float32(0.5)
    p = p * r + jnp.float32(1.0)
    p = p * r + jnp.float32(1.0)
    scale = lax.bitcast_convert_type((k + 127) << 23, jnp.float32)
    return p * scale


def _sc_body(logits, wb, out, buf0, buf1, tst, taub, zbuf, wbv, obuf,
             rbf, rbi, sem0, sem1):
    c = lax.axis_index("c")
    s = lax.axis_index("s")
    wid = s * 2 + c
    row0 = wid * _RPW
    neg = jnp.float32(-jnp.inf)
    negv = jnp.full((16,), neg, jnp.float32)
    iota = lax.iota(jnp.int32, 16)

    pltpu.sync_copy(wb, wbv)
    wv = wbv[...]
    w = [wv[i] for i in range(5)]
    bias = wv[5]

    def src(t):
        g = t // _NC
        cc = t % _NC
        return logits.at[pl.ds(row0 + g * 8, 8), pl.ds(cc * _CH, _CH)]

    pltpu.async_copy(src(0), buf0, sem0)

    def consume(t, buf, sem, nbuf, nsem):
        pltpu.make_async_copy(src(t), buf, sem).wait()

        @pl.when(t + 1 < _NT)
        def _():
            pltpu.async_copy(src(t + 1), nbuf, nsem)

        g = t // _NC
        cc = t % _NC
        is_first = cc == 0
        is_last = cc == _NC - 1

        def row_body(r, _):
            @pl.when(is_first)
            def _():
                for i in range(5):
                    tst[r, i] = negv
                taub[r] = negv

            def emit_block(nvec, base):
                a = [buf[r, pl.ds(base + j * 16, 16)] for j in range(4)]
                for j in range(4, nvec):
                    a[j & 3] = jnp.maximum(
                        a[j & 3], buf[r, pl.ds(base + j * 16, 16)])
                am = jnp.maximum(jnp.maximum(a[0], a[1]),
                                 jnp.maximum(a[2], a[3]))
                m = _bmax(am, rbf)
                tau = taub[r]

                @pl.when(m[0] > tau[0])
                def _():
                    T = [tst[r, i] for i in range(5)]
                    for j in range(nvec):
                        v = buf[r, pl.ds(base + j * 16, 16)]
                        h = jnp.maximum(T[0], v)
                        v = jnp.minimum(T[0], v)
                        T[0] = h
                        h = jnp.maximum(T[1], v)
                        v = jnp.minimum(T[1], v)
                        T[1] = h
                        h = jnp.maximum(T[2], v)
                        v = jnp.minimum(T[2], v)
                        T[2] = h
                        h = jnp.maximum(T[3], v)
                        v = jnp.minimum(T[3], v)
                        T[3] = h
                        T[4] = jnp.maximum(T[4], v)
                    for i in range(5):
                        tst[r, i] = T[i]
                    U = list(T)
                    mk = negv
                    for k in range(5):
                        mk, U = _pop_max(U, iota, neg, rbf, rbi)
                    taub[r] = mk

            def blk(b, carry):
                emit_block(_VPB, b * _VPB * 16)
                return carry

            nb = jnp.where(is_last, _NB - 1, _NB)
            lax.fori_loop(0, nb, blk, 0)

            @pl.when(is_last)
            def _():
                emit_block(_TVPB, (_NB - 1) * _VPB * 16)

            @pl.when(is_last)
            def _():
                U = [tst[r, i] for i in range(5)]
                z = bias
                for k in range(5):
                    mk, U = _pop_max(U, iota, neg, rbf, rbi)
                    z = z + w[k] * mk[0]
                zbuf[g * 8 + r] = jnp.full((16,), z, jnp.float32)

            return 0

        lax.fori_loop(0, 8, row_body, 0)

    def pair(i, carry):
        consume(2 * i, buf0, sem0, buf1, sem1)
        consume(2 * i + 1, buf1, sem1, buf0, sem0)
        return carry

    lax.fori_loop(0, _NT // 2, pair, 0)

    one = jnp.float32(1.0)
    for h in range(_RPW // 16):
        o = jnp.full((16,), jnp.float32(0.0))
        for k in range(16):
            o = jnp.where(iota == k, zbuf[h * 16 + k], o)
        obuf[pl.ds(h * 16, 16)] = one / (one + _expf(-o))
    pltpu.sync_copy(obuf, out.at[pl.ds(row0, _RPW)])


_mesh = plsc.VectorSubcoreMesh(core_axis_name="c", subcore_axis_name="s")

_sc_call = functools.partial(
    pl.kernel,
    out_type=jax.ShapeDtypeStruct((_ROWS,), jnp.float32),
    mesh=_mesh,
    scratch_types=[
        pltpu.VMEM((8, _CH), jnp.float32),
        pltpu.VMEM((8, _CH), jnp.float32),
        pltpu.VMEM((8, 5, 16), jnp.float32),
        pltpu.VMEM((8, 16), jnp.float32),
        pltpu.VMEM((_RPW, 16), jnp.float32),
        pltpu.VMEM((16,), jnp.float32),
        pltpu.VMEM((_RPW,), jnp.float32),
        pltpu.VMEM((32,), jnp.float32),
        pltpu.VMEM((32,), jnp.int32),
        pltpu.SemaphoreType.DMA,
        pltpu.SemaphoreType.DMA,
    ],
)(_sc_body)


def kernel(logits, features, W, b):
    del features
    wb = jnp.zeros((16,), jnp.float32).at[:5].set(W[0]).at[5].set(b[0])
    out = _sc_call(logits, wb)
    return out.reshape(_ROWS, 1)


# speedup vs baseline: 1.9981x; 1.9981x over previous
import functools
import jax
import jax.numpy as jnp
from jax import lax
from jax.experimental import pallas as pl
from jax.experimental.pallas import tpu as pltpu
from jax.experimental.pallas import tpu_sc as plsc

_ROWS = 1024
_COLS = 100000
_SC_ROWS = 256
_NW = 32
_RPW = _SC_ROWS // _NW
_NG = _RPW // 8
_CH = 5888
_NC = 17
_VPB = 46
_NB = 8
_TVPB = 40
_NT = _NG * _NC

_R = 256
_C = 4096


def _bmax(x, rbf):
    for sh in (8, 4, 2, 1):
        rbf[pl.ds(0, 16)] = x
        rbf[pl.ds(16, 16)] = x
        x = jnp.maximum(x, rbf[pl.ds(sh, 16)])
    return x


def _bmin(x, rbi):
    for sh in (8, 4, 2, 1):
        rbi[pl.ds(0, 16)] = x
        rbi[pl.ds(16, 16)] = x
        x = jnp.minimum(x, rbi[pl.ds(sh, 16)])
    return x


def _pop_max(U, iota, neg, rbf, rbi):
    m = _bmax(U[0], rbf)
    ic = jnp.where(U[0] == m, iota, 16)
    f = _bmin(ic, rbi)
    hit = iota == f
    U = [jnp.where(hit, U[1], U[0]),
         jnp.where(hit, U[2], U[1]),
         jnp.where(hit, U[3], U[2]),
         jnp.where(hit, U[4], U[3]),
         jnp.where(hit, neg, U[4])]
    return m, U


def _expf(x):
    x = jnp.minimum(jnp.maximum(x, jnp.float32(-87.0)), jnp.float32(87.0))
    t = x * jnp.float32(1.4426950408889634)
    magic = jnp.float32(12582912.0)
    fm = t + magic
    k = lax.bitcast_convert_type(fm, jnp.int32) - jnp.int32(0x4B400000)
    tk = fm - magic
    r = (x - tk * jnp.float32(0.693359375)) - tk * jnp.float32(-2.12194440e-4)
    p = jnp.float32(1.0 / 720.0)
    p = p * r + jnp.float32(1.0 / 120.0)
    p = p * r + jnp.float32(1.0 / 24.0)
    p = p * r + jnp.float32(1.0 / 6.0)
    p = p * r + jnp.float32(0.5)
    p = p * r + jnp.float32(1.0)
    p = p * r + jnp.float32(1.0)
    scale = lax.bitcast_convert_type((k + 127) << 23, jnp.float32)
    return p * scale


def _sc_body(logits, wb, out, buf0, buf1, tst, taub, zbuf, wbv, obuf,
             rbf, rbi, sem0, sem1):
    c = lax.axis_index("c")
    s = lax.axis_index("s")
    wid = s * 2 + c
    row0 = wid * _RPW
    neg = jnp.float32(-jnp.inf)
    negv = jnp.full((16,), neg, jnp.float32)
    iota = lax.iota(jnp.int32, 16)

    pltpu.sync_copy(wb, wbv)
    wv = wbv[...]
    w = [wv[i] for i in range(5)]
    bias = wv[5]

    def src(t):
        g = t // _NC
        cc = t % _NC
        return logits.at[pl.ds(row0 + g * 8, 8), pl.ds(cc * _CH, _CH)]

    pltpu.async_copy(src(0), buf0, sem0)

    def consume(t, buf, sem, nbuf, nsem):
        pltpu.make_async_copy(src(t), buf, sem).wait()

        @pl.when(t + 1 < _NT)
        def _():
            pltpu.async_copy(src(t + 1), nbuf, nsem)

        g = t // _NC
        cc = t % _NC
        is_first = cc == 0
        is_last = cc == _NC - 1

        def row_body(r, _):
            @pl.when(is_first)
            def _():
                for i in range(5):
                    tst[r, i] = negv
                taub[r] = negv

            def emit_block(nvec, base):
                a = [buf[r, pl.ds(base + j * 16, 16)] for j in range(4)]
                for j in range(4, nvec):
                    a[j & 3] = jnp.maximum(
                        a[j & 3], buf[r, pl.ds(base + j * 16, 16)])
                am = jnp.maximum(jnp.maximum(a[0], a[1]),
                                 jnp.maximum(a[2], a[3]))
                m = _bmax(am, rbf)
                tau = taub[r]

                @pl.when(m[0] > tau[0])
                def _():
                    T = [tst[r, i] for i in range(5)]
                    for j in range(nvec):
                        v = buf[r, pl.ds(base + j * 16, 16)]
                        h = jnp.maximum(T[0], v)
                        v = jnp.minimum(T[0], v)
                        T[0] = h
                        h = jnp.maximum(T[1], v)
                        v = jnp.minimum(T[1], v)
                        T[1] = h
                        h = jnp.maximum(T[2], v)
                        v = jnp.minimum(T[2], v)
                        T[2] = h
                        h = jnp.maximum(T[3], v)
                        v = jnp.minimum(T[3], v)
                        T[3] = h
                        T[4] = jnp.maximum(T[4], v)
                    for i in range(5):
                        tst[r, i] = T[i]
                    U = list(T)
                    mk = negv
                    for k in range(5):
                        mk, U = _pop_max(U, iota, neg, rbf, rbi)
                    taub[r] = mk

            def blk(b, carry):
                emit_block(_VPB, b * _VPB * 16)
                return carry

            nb = jnp.where(is_last, _NB - 1, _NB)
            lax.fori_loop(0, nb, blk, 0)

            @pl.when(is_last)
            def _():
                emit_block(_TVPB, (_NB - 1) * _VPB * 16)

            @pl.when(is_last)
            def _():
                U = [tst[r, i] for i in range(5)]
                z = bias
                for k in range(5):
                    mk, U = _pop_max(U, iota, neg, rbf, rbi)
                    z = z + w[k] * mk[0]
                zbuf[g * 8 + r] = jnp.full((16,), z, jnp.float32)

            return 0

        lax.fori_loop(0, 8, row_body, 0)

    def pair(i, carry):
        consume(2 * i, buf0, sem0, buf1, sem1)
        consume(2 * i + 1, buf1, sem1, buf0, sem0)
        return carry

    lax.fori_loop(0, _NT // 2, pair, 0)
    if _NT % 2:
        def last(i, carry):
            consume(i, buf0, sem0, buf1, sem1)
            return carry
        lax.fori_loop(_NT - 1, _NT, last, 0)

    one = jnp.float32(1.0)
    o = jnp.full((16,), jnp.float32(0.0))
    for k in range(_RPW):
        o = jnp.where(iota == k, zbuf[k], o)
    obuf[pl.ds(0, 16)] = one / (one + _expf(-o))
    pltpu.sync_copy(obuf.at[pl.ds(0, _RPW)], out.at[pl.ds(row0, _RPW)])


_mesh = plsc.VectorSubcoreMesh(core_axis_name="c", subcore_axis_name="s")

_sc_call = functools.partial(
    pl.kernel,
    out_type=jax.ShapeDtypeStruct((_SC_ROWS,), jnp.float32),
    mesh=_mesh,
    scratch_types=[
        pltpu.VMEM((8, _CH), jnp.float32),
        pltpu.VMEM((8, _CH), jnp.float32),
        pltpu.VMEM((8, 5, 16), jnp.float32),
        pltpu.VMEM((8, 16), jnp.float32),
        pltpu.VMEM((_RPW, 16), jnp.float32),
        pltpu.VMEM((16,), jnp.float32),
        pltpu.VMEM((16,), jnp.float32),
        pltpu.VMEM((32,), jnp.float32),
        pltpu.VMEM((32,), jnp.int32),
        pltpu.SemaphoreType.DMA,
        pltpu.SemaphoreType.DMA,
    ],
)(_sc_body)


def _topk_body(w_ref, b_ref, x_ref, o_ref, t_ref, *, nj, blk_c, n_cols):
    j = pl.program_id(1)

    @pl.when(j == 0)
    def _init():
        t_ref[...] = jnp.full_like(t_ref, -jnp.inf)

    base = j * blk_c
    lane = jax.lax.broadcasted_iota(jnp.int32, (t_ref.shape[1], 128), 1)

    T = [t_ref[k] for k in range(5)]
    for c in range(blk_c // 128):
        v = x_ref[:, c * 128:(c + 1) * 128]
        v = jnp.where(base + c * 128 + lane < n_cols, v, -jnp.inf)
        for k in range(5):
            hi = jnp.maximum(T[k], v)
            v = jnp.minimum(T[k], v)
            T[k] = hi
    for k in range(5):
        t_ref[k] = T[k]

    @pl.when(j == nj - 1)
    def _final():
        Tf = [t_ref[k] for k in range(5)]
        z = b_ref[0]
        for k in range(5):
            m = jnp.max(Tf[0], axis=1, keepdims=True)
            eq = Tf[0] == m
            cand = jnp.where(eq, lane, 1 << 20)
            jmin = jnp.min(cand, axis=1, keepdims=True)
            hit = lane == jmin
            Tf[0] = jnp.where(hit, Tf[1], Tf[0])
            Tf[1] = jnp.where(hit, Tf[2], Tf[1])
            Tf[2] = jnp.where(hit, Tf[3], Tf[2])
            Tf[3] = jnp.where(hit, Tf[4], Tf[3])
            Tf[4] = jnp.where(hit, -jnp.inf, Tf[4])
            z = z + w_ref[0, k] * m
        o_ref[...] = jax.nn.sigmoid(z)


def _tc_call(logits, W, b):
    tc_rows = _ROWS - _SC_ROWS
    row_blk0 = _SC_ROWS // _R
    nj = -(-_COLS // _C)
    body = functools.partial(_topk_body, nj=nj, blk_c=_C, n_cols=_COLS)
    return pl.pallas_call(
        body,
        grid=(tc_rows // _R, nj),
        in_specs=[
            pl.BlockSpec(memory_space=pltpu.SMEM),
            pl.BlockSpec(memory_space=pltpu.SMEM),
            pl.BlockSpec((_R, _C), lambda i, j: (i + row_blk0, j)),
        ],
        out_specs=pl.BlockSpec((_R, 1), lambda i, j: (i, 0)),
        out_shape=jax.ShapeDtypeStruct((tc_rows, 1), jnp.float32),
        scratch_shapes=[pltpu.VMEM((5, _R, 128), jnp.float32)],
        compiler_params=pltpu.CompilerParams(
            dimension_semantics=("parallel", "arbitrary"),
        ),
    )(W, b, logits)


def kernel(logits, features, W, b):
    del features
    wb = jnp.zeros((16,), jnp.float32).at[:5].set(W[0]).at[5].set(b[0])
    out_sc = _sc_call(logits, wb)
    out_tc = _tc_call(logits, W, b)
    return jnp.concatenate([out_sc.reshape(_SC_ROWS, 1), out_tc], axis=0)
